# baseline (speedup 1.0000x reference)
import functools

import jax
import jax.numpy as jnp
from jax import lax
from jax.experimental import pallas as pl
from jax.experimental.pallas import tpu as pltpu
from jax.experimental.pallas import tpu_sc as plsc

N_NODES = 10000
N_EDGES = 320000
D = 128
C = 10

NC = 2
NS = 16
NW = NC * NS
B = 125
EPT = 10000
NBT = EPT // B
SB = 16
SBP = SB // 2
NSB = NBT // SB
E_PAD = NW * EPT
N_PAD = 10240
RPT = N_PAD // NS
DUMP_ROW = N_PAD - 1
ZCH = 80
NZ = RPT // ZCH


def _sc_aggregate(x, src3, dst3):
  mesh = plsc.VectorSubcoreMesh(core_axis_name="c", subcore_axis_name="s")

  @functools.partial(
      pl.kernel,
      out_type=jax.ShapeDtypeStruct((NC, N_PAD, D), jnp.float32),
      mesh=mesh,
      scratch_types=[
          pltpu.VMEM((SB, B), jnp.int32),
          pltpu.VMEM((SB, B), jnp.int32),
          pltpu.VMEM((SB, B), jnp.int32),
          pltpu.VMEM((SB, B), jnp.int32),
          pltpu.VMEM((B, D), jnp.float32),
          pltpu.VMEM((B, D), jnp.float32),
          pltpu.VMEM_SHARED((N_PAD, D), jnp.float32),
          pltpu.SemaphoreType.DMA,
          pltpu.SemaphoreType.DMA,
          pltpu.SemaphoreType.DMA,
          pltpu.SemaphoreType.DMA,
          pltpu.SemaphoreType.DMA,
      ],
  )
  def agg_kernel(x_hbm, src_hbm, dst_hbm, out_hbm,
                 src_a, src_b, dst_a, dst_b, buf0, buf1, acc_sh,
                 gsem0, gsem1, isem, ssem0, ssem1):
    c = lax.axis_index("c")
    s = lax.axis_index("s")
    wid = c * NS + s

    slots = [(src_a, dst_a), (src_b, dst_b)]
    pltpu.async_copy(src_hbm.at[wid, 0], src_a, isem)
    pltpu.async_copy(dst_hbm.at[wid, 0], dst_a, isem)

    zero = jnp.zeros((16,), jnp.float32)

    def zrow(i, carry):
      for k in range(D // 16):
        buf0[i, pl.ds(k * 16, 16)] = zero
      return carry

    lax.fori_loop(0, ZCH, zrow, 0)
    for z in range(NZ):
      pltpu.async_copy(buf0.at[pl.ds(0, ZCH)],
                       acc_sh.at[pl.ds(s * RPT + z * ZCH, ZCH)], gsem0)
    for z in range(NZ):
      pltpu.make_async_copy(buf0.at[pl.ds(0, ZCH)],
                            acc_sh.at[pl.ds(s * RPT + z * ZCH, ZCH)],
                            gsem0).wait()
    plsc.subcore_barrier()

    pltpu.make_async_copy(src_hbm.at[wid, 0], src_a, isem).wait()
    pltpu.make_async_copy(dst_hbm.at[wid, 0], dst_a, isem).wait()
    pltpu.async_copy(src_hbm.at[wid, 1], src_b, isem)
    pltpu.async_copy(dst_hbm.at[wid, 1], dst_b, isem)
    pltpu.async_copy(x_hbm.at[src_a.at[0]], buf0, gsem0)
    pltpu.async_copy(x_hbm.at[src_a.at[1]], buf1, gsem1)

    for sb in range(NSB):
      sa, da = slots[sb % 2]

      def pair(bb, carry, sa=sa, da=da):
        j0 = 2 * bb
        pltpu.make_async_copy(x_hbm.at[sa.at[j0]], buf0, gsem0).wait()
        pltpu.async_copy(buf0, acc_sh.at[da.at[j0]], ssem0, add=True)
        pltpu.make_async_copy(x_hbm.at[sa.at[j0 + 1]], buf1, gsem1).wait()
        pltpu.async_copy(buf1, acc_sh.at[da.at[j0 + 1]], ssem1, add=True)
        pltpu.make_async_copy(buf0, acc_sh.at[da.at[j0]], ssem0).wait()
        pltpu.async_copy(x_hbm.at[sa.at[j0 + 2]], buf0, gsem0)
        pltpu.make_async_copy(buf1, acc_sh.at[da.at[j0 + 1]], ssem1).wait()
        pltpu.async_copy(x_hbm.at[sa.at[j0 + 3]], buf1, gsem1)
        return carry

      lax.fori_loop(0, SBP - 1, pair, 0)

      pltpu.make_async_copy(x_hbm.at[sa.at[SB - 2]], buf0, gsem0).wait()
      pltpu.async_copy(buf0, acc_sh.at[da.at[SB - 2]], ssem0, add=True)
      if sb + 1 < NSB:
        sn, dn = slots[(sb + 1) % 2]
        pltpu.make_async_copy(src_hbm.at[wid, sb + 1], sn, isem).wait()
        pltpu.make_async_copy(dst_hbm.at[wid, sb + 1], dn, isem).wait()
      pltpu.make_async_copy(x_hbm.at[sa.at[SB - 1]], buf1, gsem1).wait()
      pltpu.async_copy(buf1, acc_sh.at[da.at[SB - 1]], ssem1, add=True)
      pltpu.make_async_copy(buf0, acc_sh.at[da.at[SB - 2]], ssem0).wait()
      pltpu.make_async_copy(buf1, acc_sh.at[da.at[SB - 1]], ssem1).wait()
      if sb + 1 < NSB:
        pltpu.async_copy(x_hbm.at[sn.at[0]], buf0, gsem0)
        pltpu.async_copy(x_hbm.at[sn.at[1]], buf1, gsem1)
        if sb + 2 < NSB:
          pltpu.async_copy(src_hbm.at[wid, sb + 2], sa, isem)
          pltpu.async_copy(dst_hbm.at[wid, sb + 2], da, isem)

    plsc.subcore_barrier()
    pltpu.sync_copy(acc_sh.at[pl.ds(s * RPT, RPT)],
                    out_hbm.at[c, pl.ds(s * RPT, RPT)])

  return agg_kernel(x, src3, dst3)


_BLK = 2000


def _mlp_layer(x, p, Wa, ba, Wb, bb):
  def body(x_ref, p_ref, wa, ba_, wb, bb_, y_ref):
    h = x_ref[...] + p_ref[0] + p_ref[1]
    a = jnp.maximum(
        jnp.dot(h, wa[...], preferred_element_type=jnp.float32) + ba_[...], 0.0)
    y = jnp.dot(a, wb[...], preferred_element_type=jnp.float32) + bb_[...]
    y_ref[...] = jnp.maximum(y, 0.0)

  row = lambda i: (i, 0)
  full = lambda i: (0, 0)
  return pl.pallas_call(
      body,
      grid=(N_NODES // _BLK,),
      in_specs=[
          pl.BlockSpec((_BLK, D), row),
          pl.BlockSpec((NC, _BLK, D), lambda i: (0, i, 0)),
          pl.BlockSpec((D, D), full),
          pl.BlockSpec((1, D), full),
          pl.BlockSpec((D, D), full),
          pl.BlockSpec((1, D), full),
      ],
      out_specs=pl.BlockSpec((_BLK, D), row),
      out_shape=jax.ShapeDtypeStruct((N_NODES, D), jnp.float32),
  )(x, p, Wa, ba.reshape(1, D), Wb, bb.reshape(1, D))


def _mlp_layer3_head(x, p, Wa, ba, Wb, bb, Wc, bc, Wfp, bfp):
  grid_n = N_NODES // _BLK

  def body(x_ref, p_ref, wa, ba_, wb, bb_, wc, bc_, wf, bf_, o_ref, acc):
    i = pl.program_id(0)
    h = x_ref[...] + p_ref[0] + p_ref[1]
    a = jnp.maximum(
        jnp.dot(h, wa[...], preferred_element_type=jnp.float32) + ba_[...], 0.0)
    y = jnp.maximum(
        jnp.dot(a, wb[...], preferred_element_type=jnp.float32) + bb_[...], 0.0)
    colsum = jnp.sum(y, axis=0, keepdims=True)

    @pl.when(i == 0)
    def _():
      acc[...] = colsum

    @pl.when(i > 0)
    def _():
      acc[...] = acc[...] + colsum

    @pl.when(i == grid_n - 1)
    def _():
      pooled = acc[...] * (1.0 / N_NODES)
      r = jnp.dot(pooled, wc[...], preferred_element_type=jnp.float32) + bc_[...]
      o_ref[...] = jnp.dot(r, wf[...], preferred_element_type=jnp.float32) + bf_[...]

  row = lambda i: (i, 0)
  full = lambda i: (0, 0)
  return pl.pallas_call(
      body,
      grid=(grid_n,),
      in_specs=[
          pl.BlockSpec((_BLK, D), row),
          pl.BlockSpec((NC, _BLK, D), lambda i: (0, i, 0)),
          pl.BlockSpec((D, D), full),
          pl.BlockSpec((1, D), full),
          pl.BlockSpec((D, D), full),
          pl.BlockSpec((1, D), full),
          pl.BlockSpec((D, D), full),
          pl.BlockSpec((1, D), full),
          pl.BlockSpec((D, D), full),
          pl.BlockSpec((1, D), full),
      ],
      out_specs=pl.BlockSpec((1, D), full),
      out_shape=jax.ShapeDtypeStruct((1, D), jnp.float32),
      scratch_shapes=[pltpu.VMEM((1, D), jnp.float32)],
  )(x, p, Wa, ba.reshape(1, D), Wb, bb.reshape(1, D),
    Wc, bc.reshape(1, D), Wfp, bfp.reshape(1, D))


def kernel(x, edge_index, W1a, b1a, W1b, b1b, W2a, b2a, W2b, b2b,
           W3a, b3a, W3b, b3b, Wc, bc, Wf, bf):
  ei = edge_index.astype(jnp.int32)
  pad_n = E_PAD - N_EDGES
  src3 = jnp.concatenate(
      [ei[0], jnp.zeros((pad_n,), jnp.int32)]).reshape(NW, NSB, SB, B)
  pad_dst = N_NODES + (jnp.arange(pad_n, dtype=jnp.int32) % (N_PAD - N_NODES))
  dst3 = jnp.concatenate([ei[1], pad_dst]).reshape(NW, NSB, SB, B)

  Wfp = jnp.zeros((D, D), jnp.float32).at[:, :C].set(Wf)
  bfp = jnp.zeros((D,), jnp.float32).at[:C].set(bf)

  h = x
  p = _sc_aggregate(h, src3, dst3)
  h = _mlp_layer(h, p, W1a, b1a, W1b, b1b)
  p = _sc_aggregate(h, src3, dst3)
  h = _mlp_layer(h, p, W2a, b2a, W2b, b2b)
  p = _sc_aggregate(h, src3, dst3)
  out = _mlp_layer3_head(h, p, W3a, b3a, W3b, b3b, Wc, bc, Wfp, bfp)

  return (out[:, :C], edge_index)


# speedup vs baseline: 1.3011x; 1.3011x over previous
import functools

import jax
import jax.numpy as jnp
from jax import lax
from jax.experimental import pallas as pl
from jax.experimental.pallas import tpu as pltpu
from jax.experimental.pallas import tpu_sc as plsc

N_NODES = 10000
N_EDGES = 320000
D = 128
C = 10

NC = 2
NS = 16
NW = NC * NS
B = 125
EPT = 10000
NBT = EPT // B
SB = 16
SBP = SB // 2
NSB = NBT // SB
E_PAD = NW * EPT
N_PAD = 10240
RPT = N_PAD // NS
DUMP_ROW = N_PAD - 1
ZCH = 80
NZ = RPT // ZCH


def _sc_aggregate(x, src3, dst3):
  mesh = plsc.VectorSubcoreMesh(core_axis_name="c", subcore_axis_name="s")

  @functools.partial(
      pl.kernel,
      out_type=jax.ShapeDtypeStruct((NC, N_PAD, D), jnp.float32),
      mesh=mesh,
      scratch_types=[
          pltpu.VMEM((SB, B), jnp.int32),
          pltpu.VMEM((SB, B), jnp.int32),
          pltpu.VMEM((SB, B), jnp.int32),
          pltpu.VMEM((SB, B), jnp.int32),
          pltpu.VMEM((B, D), jnp.float32),
          pltpu.VMEM((B, D), jnp.float32),
          pltpu.VMEM_SHARED((N_PAD, D), jnp.float32),
          pltpu.SemaphoreType.DMA,
          pltpu.SemaphoreType.DMA,
          pltpu.SemaphoreType.DMA,
      ],
  )
  def agg_kernel(x_hbm, src_hbm, dst_hbm, out_hbm,
                 src_a, src_b, dst_a, dst_b, buf0, buf1, acc_sh,
                 gsem0, gsem1, isem):
    c = lax.axis_index("c")
    s = lax.axis_index("s")
    wid = c * NS + s

    slots = [(src_a, dst_a), (src_b, dst_b)]
    pltpu.async_copy(src_hbm.at[wid, 0], src_a, isem)
    pltpu.async_copy(dst_hbm.at[wid, 0], dst_a, isem)

    zero = jnp.zeros((16,), jnp.float32)

    def zrow(i, carry):
      for k in range(D // 16):
        buf0[i, pl.ds(k * 16, 16)] = zero
      return carry

    lax.fori_loop(0, ZCH, zrow, 0)
    for z in range(NZ):
      pltpu.async_copy(buf0.at[pl.ds(0, ZCH)],
                       acc_sh.at[pl.ds(s * RPT + z * ZCH, ZCH)], gsem0)
    for z in range(NZ):
      pltpu.make_async_copy(buf0.at[pl.ds(0, ZCH)],
                            acc_sh.at[pl.ds(s * RPT + z * ZCH, ZCH)],
                            gsem0).wait()
    plsc.subcore_barrier()

    pltpu.make_async_copy(src_hbm.at[wid, 0], src_a, isem).wait()
    pltpu.make_async_copy(dst_hbm.at[wid, 0], dst_a, isem).wait()
    pltpu.async_copy(src_hbm.at[wid, 1], src_b, isem)
    pltpu.async_copy(dst_hbm.at[wid, 1], dst_b, isem)
    pltpu.async_copy(x_hbm.at[src_a.at[0]], buf0, gsem0)
    pltpu.async_copy(x_hbm.at[src_a.at[1]], buf1, gsem1)

    for sb in range(NSB):
      sa, da = slots[sb % 2]

      def pair(bb, carry, sa=sa, da=da):
        j0 = 2 * bb
        pltpu.make_async_copy(x_hbm.at[sa.at[j0]], buf0, gsem0).wait()
        pltpu.sync_copy(buf0, acc_sh.at[da.at[j0]], add=True)
        pltpu.async_copy(x_hbm.at[sa.at[j0 + 2]], buf0, gsem0)
        pltpu.make_async_copy(x_hbm.at[sa.at[j0 + 1]], buf1, gsem1).wait()
        pltpu.sync_copy(buf1, acc_sh.at[da.at[j0 + 1]], add=True)
        pltpu.async_copy(x_hbm.at[sa.at[j0 + 3]], buf1, gsem1)
        return carry

      lax.fori_loop(0, SBP - 1, pair, 0)

      pltpu.make_async_copy(x_hbm.at[sa.at[SB - 2]], buf0, gsem0).wait()
      pltpu.sync_copy(buf0, acc_sh.at[da.at[SB - 2]], add=True)
      if sb + 1 < NSB:
        sn, dn = slots[(sb + 1) % 2]
        pltpu.make_async_copy(src_hbm.at[wid, sb + 1], sn, isem).wait()
        pltpu.make_async_copy(dst_hbm.at[wid, sb + 1], dn, isem).wait()
        pltpu.async_copy(x_hbm.at[sn.at[0]], buf0, gsem0)
      pltpu.make_async_copy(x_hbm.at[sa.at[SB - 1]], buf1, gsem1).wait()
      pltpu.sync_copy(buf1, acc_sh.at[da.at[SB - 1]], add=True)
      if sb + 1 < NSB:
        pltpu.async_copy(x_hbm.at[sn.at[1]], buf1, gsem1)
        if sb + 2 < NSB:
          pltpu.async_copy(src_hbm.at[wid, sb + 2], sa, isem)
          pltpu.async_copy(dst_hbm.at[wid, sb + 2], da, isem)

    plsc.subcore_barrier()
    pltpu.sync_copy(acc_sh.at[pl.ds(s * RPT, RPT)],
                    out_hbm.at[c, pl.ds(s * RPT, RPT)])

  return agg_kernel(x, src3, dst3)


_BLK = 2000


def _mlp_layer(x, p, Wa, ba, Wb, bb):
  def body(x_ref, p_ref, wa, ba_, wb, bb_, y_ref):
    h = x_ref[...] + p_ref[0] + p_ref[1]
    a = jnp.maximum(
        jnp.dot(h, wa[...], preferred_element_type=jnp.float32) + ba_[...], 0.0)
    y = jnp.dot(a, wb[...], preferred_element_type=jnp.float32) + bb_[...]
    y_ref[...] = jnp.maximum(y, 0.0)

  row = lambda i: (i, 0)
  full = lambda i: (0, 0)
  return pl.pallas_call(
      body,
      grid=(N_NODES // _BLK,),
      in_specs=[
          pl.BlockSpec((_BLK, D), row),
          pl.BlockSpec((NC, _BLK, D), lambda i: (0, i, 0)),
          pl.BlockSpec((D, D), full),
          pl.BlockSpec((1, D), full),
          pl.BlockSpec((D, D), full),
          pl.BlockSpec((1, D), full),
      ],
      out_specs=pl.BlockSpec((_BLK, D), row),
      out_shape=jax.ShapeDtypeStruct((N_NODES, D), jnp.float32),
  )(x, p, Wa, ba.reshape(1, D), Wb, bb.reshape(1, D))


def _mlp_layer3_head(x, p, Wa, ba, Wb, bb, Wc, bc, Wfp, bfp):
  grid_n = N_NODES // _BLK

  def body(x_ref, p_ref, wa, ba_, wb, bb_, wc, bc_, wf, bf_, o_ref, acc):
    i = pl.program_id(0)
    h = x_ref[...] + p_ref[0] + p_ref[1]
    a = jnp.maximum(
        jnp.dot(h, wa[...], preferred_element_type=jnp.float32) + ba_[...], 0.0)
    y = jnp.maximum(
        jnp.dot(a, wb[...], preferred_element_type=jnp.float32) + bb_[...], 0.0)
    colsum = jnp.sum(y, axis=0, keepdims=True)

    @pl.when(i == 0)
    def _():
      acc[...] = colsum

    @pl.when(i > 0)
    def _():
      acc[...] = acc[...] + colsum

    @pl.when(i == grid_n - 1)
    def _():
      pooled = acc[...] * (1.0 / N_NODES)
      r = jnp.dot(pooled, wc[...], preferred_element_type=jnp.float32) + bc_[...]
      o_ref[...] = jnp.dot(r, wf[...], preferred_element_type=jnp.float32) + bf_[...]

  row = lambda i: (i, 0)
  full = lambda i: (0, 0)
  return pl.pallas_call(
      body,
      grid=(grid_n,),
      in_specs=[
          pl.BlockSpec((_BLK, D), row),
          pl.BlockSpec((NC, _BLK, D), lambda i: (0, i, 0)),
          pl.BlockSpec((D, D), full),
          pl.BlockSpec((1, D), full),
          pl.BlockSpec((D, D), full),
          pl.BlockSpec((1, D), full),
          pl.BlockSpec((D, D), full),
          pl.BlockSpec((1, D), full),
          pl.BlockSpec((D, D), full),
          pl.BlockSpec((1, D), full),
      ],
      out_specs=pl.BlockSpec((1, D), full),
      out_shape=jax.ShapeDtypeStruct((1, D), jnp.float32),
      scratch_shapes=[pltpu.VMEM((1, D), jnp.float32)],
  )(x, p, Wa, ba.reshape(1, D), Wb, bb.reshape(1, D),
    Wc, bc.reshape(1, D), Wfp, bfp.reshape(1, D))


def kernel(x, edge_index, W1a, b1a, W1b, b1b, W2a, b2a, W2b, b2b,
           W3a, b3a, W3b, b3b, Wc, bc, Wf, bf):
  ei = edge_index.astype(jnp.int32)
  pad_n = E_PAD - N_EDGES
  src3 = jnp.concatenate(
      [ei[0], jnp.zeros((pad_n,), jnp.int32)]).reshape(NW, NSB, SB, B)
  pad_dst = N_NODES + (jnp.arange(pad_n, dtype=jnp.int32) % (N_PAD - N_NODES))
  dst3 = jnp.concatenate([ei[1], pad_dst]).reshape(NW, NSB, SB, B)

  Wfp = jnp.zeros((D, D), jnp.float32).at[:, :C].set(Wf)
  bfp = jnp.zeros((D,), jnp.float32).at[:C].set(bf)

  h = x
  p = _sc_aggregate(h, src3, dst3)
  h = _mlp_layer(h, p, W1a, b1a, W1b, b1b)
  p = _sc_aggregate(h, src3, dst3)
  h = _mlp_layer(h, p, W2a, b2a, W2b, b2b)
  p = _sc_aggregate(h, src3, dst3)
  out = _mlp_layer3_head(h, p, W3a, b3a, W3b, b3b, Wc, bc, Wfp, bfp)

  return (out[:, :C], edge_index)
